# baseline (speedup 1.0000x reference)
import jax
import jax.numpy as jnp
from jax import lax
from jax.experimental import pallas as pl
from jax.experimental.pallas import tpu as pltpu
from jax.experimental.pallas import tpu_sc as plsc

NGRAMS = (2, 3, 4)
V = 100000
D = 64
B, S = 1024, 200
WPAD = 256

NC, NS, L = 2, 16, 16
NW = NC * NS
BPW = B // NW


def _hash_body(seq_ref, h2_ref, h3_ref, h4_ref):
    seq = seq_ref[...]
    bb = seq.shape[0]
    for n, href in zip(NGRAMS, (h2_ref, h3_ref, h4_ref)):
        w = S - n + 1
        h = jnp.zeros((bb, w), jnp.int32)
        for j in range(n):
            h = (h * 256 + lax.slice(seq, (0, j), (bb, j + w))) % V
        h = jnp.where(h < V // 2, 2 * h, 2 * h - (V - 1))
        hp = jnp.pad(h, ((0, 0), (0, WPAD - w)))
        href[...] = jnp.stack(
            [lax.slice(hp, (0, 0), (bb, 128)),
             lax.slice(hp, (0, 128), (bb, 256))], axis=1).reshape(2 * bb, 128)


def _hashes(byte_sequence):
    bb = 256
    grid = B // bb
    return pl.pallas_call(
        _hash_body,
        grid=(grid,),
        in_specs=[pl.BlockSpec((bb, S), lambda i: (i, 0))],
        out_specs=[pl.BlockSpec((2 * bb, 128), lambda i: (i, 0))] * 3,
        out_shape=[jax.ShapeDtypeStruct((2 * B, 128), jnp.int32)] * 3,
    )(byte_sequence)


def _proj_body(t2a_ref, t2b_ref, t3a_ref, t3b_ref, t4a_ref, t4b_ref,
               wp_ref, bp_ref, p2_ref, p3_ref, p4_ref):
    wp = wp_ref[...]

    def proj(ta, tb, k):
        w = lax.slice(wp, (0, k * D), (D, (k + 1) * D))
        dg = lambda x: lax.dot_general(x, w, (((1,), (1,)), ((), ())),
                                       preferred_element_type=jnp.float32)
        return jnp.concatenate([dg(ta), dg(tb)], axis=1)

    bp = bp_ref[...]
    bp2 = jnp.concatenate([bp, bp], axis=1)
    p2_ref[...] = proj(t2a_ref[...], t2b_ref[...], 0) + bp2
    p3_ref[...] = proj(t3a_ref[...], t3b_ref[...], 1)
    p4_ref[...] = proj(t4a_ref[...], t4b_ref[...], 2)


def _projected_tables(table_2, table_3, table_4, Wp, bp):
    rb = 1000
    grid = (V // 2) // rb
    top = pl.BlockSpec((rb, D), lambda i: (i, 0))
    bot = pl.BlockSpec((rb, D), lambda i: (i + grid, 0))
    return pl.pallas_call(
        _proj_body,
        grid=(grid,),
        in_specs=[top, bot, top, bot, top, bot,
                  pl.BlockSpec((D, 3 * D), lambda i: (0, 0)),
                  pl.BlockSpec((1, D), lambda i: (0, 0))],
        out_specs=[pl.BlockSpec((rb, 2 * D), lambda i: (i, 0))] * 3,
        out_shape=[jax.ShapeDtypeStruct((V // 2, 2 * D), jnp.float32)] * 3,
    )(table_2, table_2, table_3, table_3, table_4, table_4, Wp,
      bp.reshape(1, D))


def _sc_body(p2, p3, p4, h2, h3, h4, bp_hbm, out_hbm,
             idx2, idx3, idx4, outp, buf2, buf3, buf4, biasv, sem):
    wid = lax.axis_index("s") * NC + lax.axis_index("c")
    base = wid * BPW
    pltpu.sync_copy(h2.at[pl.ds(base, BPW)], idx2)
    pltpu.sync_copy(h3.at[pl.ds(base, BPW)], idx3)
    pltpu.sync_copy(h4.at[pl.ds(base, BPW)], idx4)
    pltpu.sync_copy(bp_hbm, biasv)
    for j in range(4):
        outp[0, pl.ds(L * j, L)] = biasv[pl.ds(L * j, L)]

    def row_step(r, carry):
        copies = (
            pltpu.async_copy(p2.at[idx2.at[r, 0]], buf2.at[pl.ds(0, 128)], sem),
            pltpu.async_copy(p2.at[idx2.at[r, 1, pl.ds(0, 72)]],
                             buf2.at[pl.ds(128, 72)], sem),
            pltpu.async_copy(p3.at[idx3.at[r, 0]], buf3.at[pl.ds(0, 128)], sem),
            pltpu.async_copy(p3.at[idx3.at[r, 1, pl.ds(0, 72)]],
                             buf3.at[pl.ds(128, 72)], sem),
            pltpu.async_copy(p4.at[idx4.at[r, 0]], buf4.at[pl.ds(0, 128)], sem),
            pltpu.async_copy(p4.at[idx4.at[r, 1, pl.ds(0, 72)]],
                             buf4.at[pl.ds(128, 72)], sem),
        )
        for c in copies:
            c.wait()

        for j in range(4):
            outp[0, pl.ds(D + L * j, L)] = (buf2[0, pl.ds(L * j, L)]
                                            + buf3[0, pl.ds(L * j, L)])

        def pair_step(t, carry2):
            for j in range(4):
                outp[t, pl.ds(L * j, L)] = (
                    buf2[2 * t - 1, pl.ds(L * j, L)]
                    + buf3[2 * t - 1, pl.ds(L * j, L)]
                    + buf4[2 * t - 2, pl.ds(L * j, L)])
            for j in range(4):
                outp[t, pl.ds(D + L * j, L)] = (
                    buf2[2 * t, pl.ds(L * j, L)]
                    + buf3[2 * t, pl.ds(L * j, L)]
                    + buf4[2 * t - 1, pl.ds(L * j, L)])
            return carry2

        lax.fori_loop(1, 99, pair_step, 0, unroll=2)
        for j in range(4):
            outp[99, pl.ds(L * j, L)] = (buf2[197, pl.ds(L * j, L)]
                                         + buf3[197, pl.ds(L * j, L)]
                                         + buf4[196, pl.ds(L * j, L)])
        for j in range(4):
            outp[99, pl.ds(D + L * j, L)] = buf2[198, pl.ds(L * j, L)]
        pltpu.sync_copy(outp.at[pl.ds(0, 100)],
                        out_hbm.at[pl.ds(100 * (base + r), 100)])
        return carry

    lax.fori_loop(0, BPW, row_step, 0)


def _sc_gather(P2, P3, P4, h2, h3, h4, bp):
    mesh = plsc.VectorSubcoreMesh(core_axis_name="c", subcore_axis_name="s",
                                  num_cores=NC, num_subcores=NS)
    f = pl.kernel(
        _sc_body,
        out_type=jax.ShapeDtypeStruct((B * 100, 128), jnp.float32),
        mesh=mesh,
        compiler_params=pltpu.CompilerParams(use_tc_tiling_on_sc=False),
        scratch_types=[
            pltpu.VMEM((BPW, 2, 128), jnp.int32),
            pltpu.VMEM((BPW, 2, 128), jnp.int32),
            pltpu.VMEM((BPW, 2, 128), jnp.int32),
            pltpu.VMEM((104, 2 * D), jnp.float32),
            pltpu.VMEM((200, D), jnp.float32),
            pltpu.VMEM((200, D), jnp.float32),
            pltpu.VMEM((200, D), jnp.float32),
            pltpu.VMEM((D,), jnp.float32),
            pltpu.SemaphoreType.DMA,
        ],
    )
    return f(P2, P3, P4, h2, h3, h4, bp)


def kernel(byte_sequence, table_2, table_3, table_4, Wp, bp):
    h2, h3, h4 = _hashes(byte_sequence)
    P2, P3, P4 = _projected_tables(table_2, table_3, table_4, Wp, bp)
    acc = _sc_gather(
        P2.reshape(V, D), P3.reshape(V, D), P4.reshape(V, D),
        h2.reshape(B, 2, 128), h3.reshape(B, 2, 128), h4.reshape(B, 2, 128),
        bp)
    return acc.reshape(B, S, D)


# speedup vs baseline: 1.1272x; 1.1272x over previous
import jax
import jax.numpy as jnp
from jax import lax
from jax.experimental import pallas as pl
from jax.experimental.pallas import tpu as pltpu
from jax.experimental.pallas import tpu_sc as plsc

NGRAMS = (2, 3, 4)
V = 100000
D = 64
B, S = 1024, 200
WPAD = 256

NC, NS, L = 2, 16, 16
NW = NC * NS
BPW = B // NW


def _hash_body(seq_ref, h2_ref, h3_ref, h4_ref):
    seq = seq_ref[...]
    bb = seq.shape[0]
    for n, href in zip(NGRAMS, (h2_ref, h3_ref, h4_ref)):
        w = S - n + 1
        h = jnp.zeros((bb, w), jnp.int32)
        for j in range(n):
            h = (h * 256 + lax.slice(seq, (0, j), (bb, j + w))) % V
        h = jnp.where(h < V // 2, 2 * h, 2 * h - (V - 1))
        hp = jnp.pad(h, ((0, 0), (0, WPAD - w)))
        href[...] = jnp.stack(
            [lax.slice(hp, (0, 0), (bb, 128)),
             lax.slice(hp, (0, 128), (bb, 256))], axis=1).reshape(2 * bb, 128)


def _hashes(byte_sequence):
    bb = 256
    grid = B // bb
    return pl.pallas_call(
        _hash_body,
        grid=(grid,),
        in_specs=[pl.BlockSpec((bb, S), lambda i: (i, 0))],
        out_specs=[pl.BlockSpec((2 * bb, 128), lambda i: (i, 0))] * 3,
        out_shape=[jax.ShapeDtypeStruct((2 * B, 128), jnp.int32)] * 3,
    )(byte_sequence)


def _proj_body(t2a_ref, t2b_ref, t3a_ref, t3b_ref, t4a_ref, t4b_ref,
               wp_ref, bp_ref, p2_ref, p3_ref, p4_ref):
    wp = wp_ref[...]

    def proj(ta, tb, k):
        w = lax.slice(wp, (0, k * D), (D, (k + 1) * D))
        dg = lambda x: lax.dot_general(x, w, (((1,), (1,)), ((), ())),
                                       preferred_element_type=jnp.float32)
        return jnp.concatenate([dg(ta), dg(tb)], axis=1)

    bp = bp_ref[...]
    bp2 = jnp.concatenate([bp, bp], axis=1)
    p2_ref[...] = proj(t2a_ref[...], t2b_ref[...], 0) + bp2
    p3_ref[...] = proj(t3a_ref[...], t3b_ref[...], 1)
    p4_ref[...] = proj(t4a_ref[...], t4b_ref[...], 2)


def _projected_tables(table_2, table_3, table_4, Wp, bp):
    rb = 1000
    grid = (V // 2) // rb
    top = pl.BlockSpec((rb, D), lambda i: (i, 0))
    bot = pl.BlockSpec((rb, D), lambda i: (i + grid, 0))
    return pl.pallas_call(
        _proj_body,
        grid=(grid,),
        in_specs=[top, bot, top, bot, top, bot,
                  pl.BlockSpec((D, 3 * D), lambda i: (0, 0)),
                  pl.BlockSpec((1, D), lambda i: (0, 0))],
        out_specs=[pl.BlockSpec((rb, 2 * D), lambda i: (i, 0))] * 3,
        out_shape=[jax.ShapeDtypeStruct((V // 2, 2 * D), jnp.float32)] * 3,
    )(table_2, table_2, table_3, table_3, table_4, table_4, Wp,
      bp.reshape(1, D))


def _sc_body(p2, p3, p4, h2, h3, h4, bp_hbm, out_hbm,
             idx2, idx3, idx4, outp, outp2, buf2, buf3, buf4,
             buf2b, buf3b, buf4b, biasv, semA, semB):
    wid = lax.axis_index("s") * NC + lax.axis_index("c")
    base = wid * BPW
    pltpu.sync_copy(h2.at[pl.ds(base, BPW)], idx2)
    pltpu.sync_copy(h3.at[pl.ds(base, BPW)], idx3)
    pltpu.sync_copy(h4.at[pl.ds(base, BPW)], idx4)
    pltpu.sync_copy(bp_hbm, biasv)
    for j in range(4):
        outp[0, pl.ds(L * j, L)] = biasv[pl.ds(L * j, L)]
        outp2[0, pl.ds(L * j, L)] = biasv[pl.ds(L * j, L)]

    def descs(r, b2, b3, b4, s):
        return (
            pltpu.make_async_copy(p2.at[idx2.at[r, 0]],
                                  b2.at[pl.ds(0, 128)], s),
            pltpu.make_async_copy(p2.at[idx2.at[r, 1, pl.ds(0, 72)]],
                                  b2.at[pl.ds(128, 72)], s),
            pltpu.make_async_copy(p3.at[idx3.at[r, 0]],
                                  b3.at[pl.ds(0, 128)], s),
            pltpu.make_async_copy(p3.at[idx3.at[r, 1, pl.ds(0, 72)]],
                                  b3.at[pl.ds(128, 72)], s),
            pltpu.make_async_copy(p4.at[idx4.at[r, 0]],
                                  b4.at[pl.ds(0, 128)], s),
            pltpu.make_async_copy(p4.at[idx4.at[r, 1, pl.ds(0, 72)]],
                                  b4.at[pl.ds(128, 72)], s),
        )

    def fire(r, b2, b3, b4, s):
        for c in descs(r, b2, b3, b4, s):
            c.start()

    def drain(r, b2, b3, b4, s):
        for c in descs(r, b2, b3, b4, s):
            c.wait()

    def alu(b2, b3, b4, op):
        for j in range(4):
            op[0, pl.ds(D + L * j, L)] = (b2[0, pl.ds(L * j, L)]
                                          + b3[0, pl.ds(L * j, L)])

        def pair_step(t, carry2):
            for j in range(4):
                op[t, pl.ds(L * j, L)] = (
                    b2[2 * t - 1, pl.ds(L * j, L)]
                    + b3[2 * t - 1, pl.ds(L * j, L)]
                    + b4[2 * t - 2, pl.ds(L * j, L)])
            for j in range(4):
                op[t, pl.ds(D + L * j, L)] = (
                    b2[2 * t, pl.ds(L * j, L)]
                    + b3[2 * t, pl.ds(L * j, L)]
                    + b4[2 * t - 1, pl.ds(L * j, L)])
            return carry2

        lax.fori_loop(1, 99, pair_step, 0, unroll=2)
        for j in range(4):
            op[99, pl.ds(L * j, L)] = (b2[197, pl.ds(L * j, L)]
                                       + b3[197, pl.ds(L * j, L)]
                                       + b4[196, pl.ds(L * j, L)])
        for j in range(4):
            op[99, pl.ds(D + L * j, L)] = b2[198, pl.ds(L * j, L)]

    def write_out(r, op):
        pltpu.sync_copy(op.at[pl.ds(0, 100)],
                        out_hbm.at[pl.ds(100 * (base + r), 100)])

    bufA = (buf2, buf3, buf4)
    bufB = (buf2b, buf3b, buf4b)
    fire(0, *bufA, semA)
    fire(1, *bufB, semB)

    def pipe_step(i, carry):
        ra = 2 * i
        drain(ra, *bufA, semA)
        alu(*bufA, outp)
        fire(lax.min(ra + 2, BPW - 1), *bufA, semA)
        write_out(ra, outp)
        drain(ra + 1, *bufB, semB)
        alu(*bufB, outp2)
        fire(lax.min(ra + 3, BPW - 1), *bufB, semB)
        write_out(ra + 1, outp2)
        return carry

    lax.fori_loop(0, BPW // 2 - 1, pipe_step, 0)
    drain(BPW - 2, *bufA, semA)
    alu(*bufA, outp)
    write_out(BPW - 2, outp)
    drain(BPW - 1, *bufB, semB)
    alu(*bufB, outp2)
    write_out(BPW - 1, outp2)


def _sc_gather(P2, P3, P4, h2, h3, h4, bp):
    mesh = plsc.VectorSubcoreMesh(core_axis_name="c", subcore_axis_name="s",
                                  num_cores=NC, num_subcores=NS)
    f = pl.kernel(
        _sc_body,
        out_type=jax.ShapeDtypeStruct((B * 100, 128), jnp.float32),
        mesh=mesh,
        compiler_params=pltpu.CompilerParams(use_tc_tiling_on_sc=False),
        scratch_types=[
            pltpu.VMEM((BPW, 2, 128), jnp.int32),
            pltpu.VMEM((BPW, 2, 128), jnp.int32),
            pltpu.VMEM((BPW, 2, 128), jnp.int32),
            pltpu.VMEM((104, 2 * D), jnp.float32),
            pltpu.VMEM((104, 2 * D), jnp.float32),
            pltpu.VMEM((200, D), jnp.float32),
            pltpu.VMEM((200, D), jnp.float32),
            pltpu.VMEM((200, D), jnp.float32),
            pltpu.VMEM((200, D), jnp.float32),
            pltpu.VMEM((200, D), jnp.float32),
            pltpu.VMEM((200, D), jnp.float32),
            pltpu.VMEM((D,), jnp.float32),
            pltpu.SemaphoreType.DMA,
            pltpu.SemaphoreType.DMA,
        ],
    )
    return f(P2, P3, P4, h2, h3, h4, bp)


def kernel(byte_sequence, table_2, table_3, table_4, Wp, bp):
    h2, h3, h4 = _hashes(byte_sequence)
    P2, P3, P4 = _projected_tables(table_2, table_3, table_4, Wp, bp)
    acc = _sc_gather(
        P2.reshape(V, D), P3.reshape(V, D), P4.reshape(V, D),
        h2.reshape(B, 2, 128), h3.reshape(B, 2, 128), h4.reshape(B, 2, 128),
        bp)
    return acc.reshape(B, S, D)


# speedup vs baseline: 1.1412x; 1.0124x over previous
import jax
import jax.numpy as jnp
from jax import lax
from jax.experimental import pallas as pl
from jax.experimental.pallas import tpu as pltpu
from jax.experimental.pallas import tpu_sc as plsc

NGRAMS = (2, 3, 4)
V = 100000
D = 64
B, S = 1024, 200
WPAD = 256

NC, NS, L = 2, 16, 16
NW = NC * NS
BPW = B // NW


def _hash_body(seq_ref, h2_ref, h3_ref, h4_ref):
    seq = seq_ref[...]
    bb = seq.shape[0]
    for n, href in zip(NGRAMS, (h2_ref, h3_ref, h4_ref)):
        w = S - n + 1
        h = jnp.zeros((bb, w), jnp.int32)
        for j in range(n):
            h = (h * 256 + lax.slice(seq, (0, j), (bb, j + w))) % V
        h = jnp.where(h < V // 2, 2 * h, 2 * h - (V - 1))
        hp = jnp.pad(h, ((0, 0), (0, WPAD - w)))
        href[...] = jnp.stack(
            [lax.slice(hp, (0, 0), (bb, 128)),
             lax.slice(hp, (0, 128), (bb, 256))], axis=1).reshape(2 * bb, 128)


def _hashes(byte_sequence):
    bb = 256
    grid = B // bb
    return pl.pallas_call(
        _hash_body,
        grid=(grid,),
        in_specs=[pl.BlockSpec((bb, S), lambda i: (i, 0))],
        out_specs=[pl.BlockSpec((2 * bb, 128), lambda i: (i, 0))] * 3,
        out_shape=[jax.ShapeDtypeStruct((2 * B, 128), jnp.int32)] * 3,
    )(byte_sequence)


def _proj_body(t2a_ref, t2b_ref, t3a_ref, t3b_ref, t4a_ref, t4b_ref,
               wp_ref, bp_ref, p2_ref, p3_ref, p4_ref):
    wp = wp_ref[...]

    def proj(ta, tb, k):
        w = lax.slice(wp, (0, k * D), (D, (k + 1) * D))
        dg = lambda x: lax.dot_general(x, w, (((1,), (1,)), ((), ())),
                                       preferred_element_type=jnp.float32)
        return jnp.concatenate([dg(ta), dg(tb)], axis=1)

    bp = bp_ref[...]
    bp2 = jnp.concatenate([bp, bp], axis=1)
    p2_ref[...] = proj(t2a_ref[...], t2b_ref[...], 0) + bp2
    p3_ref[...] = proj(t3a_ref[...], t3b_ref[...], 1)
    p4_ref[...] = proj(t4a_ref[...], t4b_ref[...], 2)


def _projected_tables(table_2, table_3, table_4, Wp, bp):
    rb = 1000
    grid = (V // 2) // rb
    top = pl.BlockSpec((rb, D), lambda i: (i, 0))
    bot = pl.BlockSpec((rb, D), lambda i: (i + grid, 0))
    return pl.pallas_call(
        _proj_body,
        grid=(grid,),
        in_specs=[top, bot, top, bot, top, bot,
                  pl.BlockSpec((D, 3 * D), lambda i: (0, 0)),
                  pl.BlockSpec((1, D), lambda i: (0, 0))],
        out_specs=[pl.BlockSpec((rb, 2 * D), lambda i: (i, 0))] * 3,
        out_shape=[jax.ShapeDtypeStruct((V // 2, 2 * D), jnp.float32)] * 3,
    )(table_2, table_2, table_3, table_3, table_4, table_4, Wp,
      bp.reshape(1, D))


def _sc_body(p2, p3, p4, h2, h3, h4, bp_hbm, out_hbm,
             idx2, idx3, idx4, outp, outp2, buf2, buf3, buf4,
             buf2b, buf3b, buf4b, biasv, semA, semB, semOA, semOB):
    wid = lax.axis_index("s") * NC + lax.axis_index("c")
    base = wid * BPW
    pltpu.sync_copy(h2.at[pl.ds(base, BPW)], idx2)
    pltpu.sync_copy(h3.at[pl.ds(base, BPW)], idx3)
    pltpu.sync_copy(h4.at[pl.ds(base, BPW)], idx4)
    pltpu.sync_copy(bp_hbm, biasv)
    for j in range(4):
        outp[0, pl.ds(L * j, L)] = biasv[pl.ds(L * j, L)]
        outp2[0, pl.ds(L * j, L)] = biasv[pl.ds(L * j, L)]

    def descs(r, b2, b3, b4, s):
        return (
            pltpu.make_async_copy(p2.at[idx2.at[r, 0]],
                                  b2.at[pl.ds(0, 128)], s),
            pltpu.make_async_copy(p2.at[idx2.at[r, 1, pl.ds(0, 72)]],
                                  b2.at[pl.ds(128, 72)], s),
            pltpu.make_async_copy(p3.at[idx3.at[r, 0]],
                                  b3.at[pl.ds(0, 128)], s),
            pltpu.make_async_copy(p3.at[idx3.at[r, 1, pl.ds(0, 72)]],
                                  b3.at[pl.ds(128, 72)], s),
            pltpu.make_async_copy(p4.at[idx4.at[r, 0]],
                                  b4.at[pl.ds(0, 128)], s),
            pltpu.make_async_copy(p4.at[idx4.at[r, 1, pl.ds(0, 72)]],
                                  b4.at[pl.ds(128, 72)], s),
        )

    def fire(r, b2, b3, b4, s):
        for c in descs(r, b2, b3, b4, s):
            c.start()

    def drain(r, b2, b3, b4, s):
        for c in descs(r, b2, b3, b4, s):
            c.wait()

    def alu(b2, b3, b4, op):
        for j in range(4):
            op[0, pl.ds(D + L * j, L)] = (b2[0, pl.ds(L * j, L)]
                                          + b3[0, pl.ds(L * j, L)])

        def pair_step(t, carry2):
            for j in range(4):
                op[t, pl.ds(L * j, L)] = (
                    b2[2 * t - 1, pl.ds(L * j, L)]
                    + b3[2 * t - 1, pl.ds(L * j, L)]
                    + b4[2 * t - 2, pl.ds(L * j, L)])
            for j in range(4):
                op[t, pl.ds(D + L * j, L)] = (
                    b2[2 * t, pl.ds(L * j, L)]
                    + b3[2 * t, pl.ds(L * j, L)]
                    + b4[2 * t - 1, pl.ds(L * j, L)])
            return carry2

        lax.fori_loop(1, 99, pair_step, 0, unroll=2)
        for j in range(4):
            op[99, pl.ds(L * j, L)] = (b2[197, pl.ds(L * j, L)]
                                       + b3[197, pl.ds(L * j, L)]
                                       + b4[196, pl.ds(L * j, L)])
        for j in range(4):
            op[99, pl.ds(D + L * j, L)] = b2[198, pl.ds(L * j, L)]

    def write_desc(r, op, s):
        return pltpu.make_async_copy(
            op.at[pl.ds(0, 100)],
            out_hbm.at[pl.ds(100 * (base + r), 100)], s)

    def write_out(r, op):
        pltpu.sync_copy(op.at[pl.ds(0, 100)],
                        out_hbm.at[pl.ds(100 * (base + r), 100)])

    bufA = (buf2, buf3, buf4)
    bufB = (buf2b, buf3b, buf4b)
    fire(0, *bufA, semA)
    fire(1, *bufB, semB)

    def pipe_step(i, carry):
        ra = 2 * i
        drain(ra, *bufA, semA)

        @pl.when(i > 0)
        def _():
            write_desc(ra, outp, semOA).wait()

        alu(*bufA, outp)
        fire(lax.min(ra + 2, BPW - 1), *bufA, semA)
        write_desc(ra, outp, semOA).start()
        drain(ra + 1, *bufB, semB)

        @pl.when(i > 0)
        def _():
            write_desc(ra + 1, outp2, semOB).wait()

        alu(*bufB, outp2)
        fire(lax.min(ra + 3, BPW - 1), *bufB, semB)
        write_desc(ra + 1, outp2, semOB).start()
        return carry

    lax.fori_loop(0, BPW // 2 - 1, pipe_step, 0)
    drain(BPW - 2, *bufA, semA)
    write_desc(BPW - 2, outp, semOA).wait()
    alu(*bufA, outp)
    write_out(BPW - 2, outp)
    drain(BPW - 1, *bufB, semB)
    write_desc(BPW - 1, outp2, semOB).wait()
    alu(*bufB, outp2)
    write_out(BPW - 1, outp2)


def _sc_gather(P2, P3, P4, h2, h3, h4, bp):
    mesh = plsc.VectorSubcoreMesh(core_axis_name="c", subcore_axis_name="s",
                                  num_cores=NC, num_subcores=NS)
    f = pl.kernel(
        _sc_body,
        out_type=jax.ShapeDtypeStruct((B * 100, 128), jnp.float32),
        mesh=mesh,
        compiler_params=pltpu.CompilerParams(use_tc_tiling_on_sc=False),
        scratch_types=[
            pltpu.VMEM((BPW, 2, 128), jnp.int32),
            pltpu.VMEM((BPW, 2, 128), jnp.int32),
            pltpu.VMEM((BPW, 2, 128), jnp.int32),
            pltpu.VMEM((104, 2 * D), jnp.float32),
            pltpu.VMEM((104, 2 * D), jnp.float32),
            pltpu.VMEM((200, D), jnp.float32),
            pltpu.VMEM((200, D), jnp.float32),
            pltpu.VMEM((200, D), jnp.float32),
            pltpu.VMEM((200, D), jnp.float32),
            pltpu.VMEM((200, D), jnp.float32),
            pltpu.VMEM((200, D), jnp.float32),
            pltpu.VMEM((D,), jnp.float32),
            pltpu.SemaphoreType.DMA,
            pltpu.SemaphoreType.DMA,
            pltpu.SemaphoreType.DMA,
            pltpu.SemaphoreType.DMA,
        ],
    )
    return f(P2, P3, P4, h2, h3, h4, bp)


def kernel(byte_sequence, table_2, table_3, table_4, Wp, bp):
    h2, h3, h4 = _hashes(byte_sequence)
    P2, P3, P4 = _projected_tables(table_2, table_3, table_4, Wp, bp)
    acc = _sc_gather(
        P2.reshape(V, D), P3.reshape(V, D), P4.reshape(V, D),
        h2.reshape(B, 2, 128), h3.reshape(B, 2, 128), h4.reshape(B, 2, 128),
        bp)
    return acc.reshape(B, S, D)


# speedup vs baseline: 1.2271x; 1.0753x over previous
import jax
import jax.numpy as jnp
from jax import lax
from jax.experimental import pallas as pl
from jax.experimental.pallas import tpu as pltpu
from jax.experimental.pallas import tpu_sc as plsc

NGRAMS = (2, 3, 4)
V = 100000
D = 64
B, S = 1024, 200
WPAD = 256

NC, NS, L = 2, 16, 16
NW = NC * NS
BPW = B // NW


def _hash_body(seq_ref, h2_ref, h3_ref, h4_ref):
    seq = seq_ref[...]
    bb = seq.shape[0]
    for n, href in zip(NGRAMS, (h2_ref, h3_ref, h4_ref)):
        w = S - n + 1
        h = jnp.zeros((bb, w), jnp.int32)
        for j in range(n):
            h = (h * 256 + lax.slice(seq, (0, j), (bb, j + w))) % V
        h = jnp.where(h < V // 2, 2 * h, 2 * h - (V - 1))
        hp = jnp.pad(h, ((0, 0), (0, WPAD - w)))
        href[...] = jnp.stack(
            [lax.slice(hp, (0, 0), (bb, 128)),
             lax.slice(hp, (0, 128), (bb, 256))], axis=1).reshape(2 * bb, 128)


def _hashes(byte_sequence):
    bb = 256
    grid = B // bb
    return pl.pallas_call(
        _hash_body,
        grid=(grid,),
        in_specs=[pl.BlockSpec((bb, S), lambda i: (i, 0))],
        out_specs=[pl.BlockSpec((2 * bb, 128), lambda i: (i, 0))] * 3,
        out_shape=[jax.ShapeDtypeStruct((2 * B, 128), jnp.int32)] * 3,
    )(byte_sequence)


def _proj_body(t2a_ref, t2b_ref, t3a_ref, t3b_ref, t4a_ref, t4b_ref,
               wp_ref, bp_ref, p2_ref, p3_ref, p4_ref):
    wp = wp_ref[...]

    def proj(ta, tb, k):
        w = lax.slice(wp, (0, k * D), (D, (k + 1) * D))
        dg = lambda x: lax.dot_general(x, w, (((1,), (1,)), ((), ())),
                                       preferred_element_type=jnp.float32)
        return jnp.concatenate([dg(ta), dg(tb)], axis=1)

    bp = bp_ref[...]
    bp2 = jnp.concatenate([bp, bp], axis=1)
    p2_ref[...] = proj(t2a_ref[...], t2b_ref[...], 0) + bp2
    p3_ref[...] = proj(t3a_ref[...], t3b_ref[...], 1)
    p4_ref[...] = proj(t4a_ref[...], t4b_ref[...], 2)


def _projected_tables(table_2, table_3, table_4, Wp, bp):
    rb = 1000
    grid = (V // 2) // rb
    top = pl.BlockSpec((rb, D), lambda i: (i, 0))
    bot = pl.BlockSpec((rb, D), lambda i: (i + grid, 0))
    return pl.pallas_call(
        _proj_body,
        grid=(grid,),
        in_specs=[top, bot, top, bot, top, bot,
                  pl.BlockSpec((D, 3 * D), lambda i: (0, 0)),
                  pl.BlockSpec((1, D), lambda i: (0, 0))],
        out_specs=[pl.BlockSpec((rb, 2 * D), lambda i: (i, 0))] * 3,
        out_shape=[jax.ShapeDtypeStruct((V // 2, 2 * D), jnp.float32)] * 3,
    )(table_2, table_2, table_3, table_3, table_4, table_4, Wp,
      bp.reshape(1, D))


def _sc_body(p2, p3, p4, h2, h3, h4, bp_hbm, out_hbm,
             idx2, idx3, idx4, outp, outp2, buf2, buf3, buf4,
             buf2b, buf3b, buf4b, biasv, semA, semB, semOA, semOB):
    wid = lax.axis_index("s") * NC + lax.axis_index("c")
    base = wid * BPW
    pltpu.sync_copy(h2.at[pl.ds(base, BPW)], idx2)
    pltpu.sync_copy(h3.at[pl.ds(base, BPW)], idx3)
    pltpu.sync_copy(h4.at[pl.ds(base, BPW)], idx4)
    pltpu.sync_copy(bp_hbm, biasv)
    for j in range(4):
        outp[0, pl.ds(L * j, L)] = biasv[pl.ds(L * j, L)]
        outp2[0, pl.ds(L * j, L)] = biasv[pl.ds(L * j, L)]

    def descs(r, b2, b3, b4, s):
        return (
            pltpu.make_async_copy(p2.at[idx2.at[r, 0]],
                                  b2.at[pl.ds(0, 128)], s),
            pltpu.make_async_copy(p2.at[idx2.at[r, 1, pl.ds(0, 72)]],
                                  b2.at[pl.ds(128, 72)], s),
            pltpu.make_async_copy(p3.at[idx3.at[r, 0]],
                                  b3.at[pl.ds(0, 128)], s),
            pltpu.make_async_copy(p3.at[idx3.at[r, 1, pl.ds(0, 72)]],
                                  b3.at[pl.ds(128, 72)], s),
            pltpu.make_async_copy(p4.at[idx4.at[r, 0]],
                                  b4.at[pl.ds(0, 128)], s),
            pltpu.make_async_copy(p4.at[idx4.at[r, 1, pl.ds(0, 72)]],
                                  b4.at[pl.ds(128, 72)], s),
        )

    def fire(r, b2, b3, b4, s):
        for c in descs(r, b2, b3, b4, s):
            c.start()

    def drain(r, b2, b3, b4, s):
        for c in descs(r, b2, b3, b4, s):
            c.wait()

    def alu(b2, b3, b4, op):
        for j in range(4):
            op[0, pl.ds(D + L * j, L)] = (b2[0, pl.ds(L * j, L)]
                                          + b3[0, pl.ds(L * j, L)])

        def pair_step(t, carry2):
            for j in range(4):
                op[t, pl.ds(L * j, L)] = (
                    b2[2 * t - 1, pl.ds(L * j, L)]
                    + b3[2 * t - 1, pl.ds(L * j, L)]
                    + b4[2 * t - 2, pl.ds(L * j, L)])
            for j in range(4):
                op[t, pl.ds(D + L * j, L)] = (
                    b2[2 * t, pl.ds(L * j, L)]
                    + b3[2 * t, pl.ds(L * j, L)]
                    + b4[2 * t - 1, pl.ds(L * j, L)])
            return carry2

        lax.fori_loop(1, 99, pair_step, 0, unroll=2)
        for j in range(4):
            op[99, pl.ds(L * j, L)] = (b2[197, pl.ds(L * j, L)]
                                       + b3[197, pl.ds(L * j, L)]
                                       + b4[196, pl.ds(L * j, L)])
        for j in range(4):
            op[99, pl.ds(D + L * j, L)] = b2[198, pl.ds(L * j, L)]

    def write_desc(r, op, s):
        return pltpu.make_async_copy(op.at[pl.ds(0, 100)],
                                     out_hbm.at[base + r], s)

    def write_out(r, op):
        pltpu.sync_copy(op.at[pl.ds(0, 100)], out_hbm.at[base + r])

    bufA = (buf2, buf3, buf4)
    bufB = (buf2b, buf3b, buf4b)
    fire(0, *bufA, semA)
    fire(1, *bufB, semB)

    def pipe_step(i, carry):
        ra = 2 * i
        drain(ra, *bufA, semA)

        @pl.when(i > 0)
        def _():
            write_desc(ra, outp, semOA).wait()

        alu(*bufA, outp)
        fire(lax.min(ra + 2, BPW - 1), *bufA, semA)
        write_desc(ra, outp, semOA).start()
        drain(ra + 1, *bufB, semB)

        @pl.when(i > 0)
        def _():
            write_desc(ra + 1, outp2, semOB).wait()

        alu(*bufB, outp2)
        fire(lax.min(ra + 3, BPW - 1), *bufB, semB)
        write_desc(ra + 1, outp2, semOB).start()
        return carry

    lax.fori_loop(0, BPW // 2 - 1, pipe_step, 0)
    drain(BPW - 2, *bufA, semA)
    write_desc(BPW - 2, outp, semOA).wait()
    alu(*bufA, outp)
    write_out(BPW - 2, outp)
    drain(BPW - 1, *bufB, semB)
    write_desc(BPW - 1, outp2, semOB).wait()
    alu(*bufB, outp2)
    write_out(BPW - 1, outp2)


def _sc_gather(P2, P3, P4, h2, h3, h4, bp):
    mesh = plsc.VectorSubcoreMesh(core_axis_name="c", subcore_axis_name="s",
                                  num_cores=NC, num_subcores=NS)
    f = pl.kernel(
        _sc_body,
        out_type=jax.ShapeDtypeStruct((B, 100, 2 * D), jnp.float32),
        mesh=mesh,
        compiler_params=pltpu.CompilerParams(use_tc_tiling_on_sc=False),
        scratch_types=[
            pltpu.VMEM((BPW, 2, 128), jnp.int32),
            pltpu.VMEM((BPW, 2, 128), jnp.int32),
            pltpu.VMEM((BPW, 2, 128), jnp.int32),
            pltpu.VMEM((104, 2 * D), jnp.float32),
            pltpu.VMEM((104, 2 * D), jnp.float32),
            pltpu.VMEM((200, D), jnp.float32),
            pltpu.VMEM((200, D), jnp.float32),
            pltpu.VMEM((200, D), jnp.float32),
            pltpu.VMEM((200, D), jnp.float32),
            pltpu.VMEM((200, D), jnp.float32),
            pltpu.VMEM((200, D), jnp.float32),
            pltpu.VMEM((D,), jnp.float32),
            pltpu.SemaphoreType.DMA,
            pltpu.SemaphoreType.DMA,
            pltpu.SemaphoreType.DMA,
            pltpu.SemaphoreType.DMA,
        ],
    )
    return f(P2, P3, P4, h2, h3, h4, bp)


def kernel(byte_sequence, table_2, table_3, table_4, Wp, bp):
    h2, h3, h4 = _hashes(byte_sequence)
    P2, P3, P4 = _projected_tables(table_2, table_3, table_4, Wp, bp)
    acc = _sc_gather(
        P2.reshape(V, D), P3.reshape(V, D), P4.reshape(V, D),
        h2.reshape(B, 2, 128), h3.reshape(B, 2, 128), h4.reshape(B, 2, 128),
        bp)
    return acc.reshape(B, S, D)


# speedup vs baseline: 1.2612x; 1.0278x over previous
import jax
import jax.numpy as jnp
from jax import lax
from jax.experimental import pallas as pl
from jax.experimental.pallas import tpu as pltpu
from jax.experimental.pallas import tpu_sc as plsc

NGRAMS = (2, 3, 4)
V = 100000
D = 64
B, S = 1024, 200
WPAD = 256

NC, NS, L = 2, 16, 16
NW = NC * NS
BPW = B // NW


def _hash_body(seq_ref, h2_ref, h3_ref, h4_ref):
    seq = seq_ref[...]
    bb = seq.shape[0]
    for n, href in zip(NGRAMS, (h2_ref, h3_ref, h4_ref)):
        w = S - n + 1
        h = jnp.zeros((bb, w), jnp.int32)
        for j in range(n):
            h = (h * 256 + lax.slice(seq, (0, j), (bb, j + w))) % V
        h = jnp.where(h < V // 2, 2 * h, 2 * h - (V - 1))
        hp = jnp.pad(h, ((0, 0), (0, WPAD - w)))
        href[...] = jnp.stack(
            [lax.slice(hp, (0, 0), (bb, 128)),
             lax.slice(hp, (0, 128), (bb, 256))], axis=1).reshape(2 * bb, 128)


def _hashes(byte_sequence):
    bb = 256
    grid = B // bb
    return pl.pallas_call(
        _hash_body,
        grid=(grid,),
        in_specs=[pl.BlockSpec((bb, S), lambda i: (i, 0))],
        out_specs=[pl.BlockSpec((2 * bb, 128), lambda i: (i, 0))] * 3,
        out_shape=[jax.ShapeDtypeStruct((2 * B, 128), jnp.int32)] * 3,
    )(byte_sequence)


def _proj_body(t2a_ref, t2b_ref, t3a_ref, t3b_ref, t4a_ref, t4b_ref,
               wp_ref, bp_ref, p2_ref, p3_ref, p4_ref):
    wp = wp_ref[...]

    def proj(ta, tb, k):
        w = lax.slice(wp, (0, k * D), (D, (k + 1) * D))
        dg = lambda x: lax.dot_general(x, w, (((1,), (1,)), ((), ())),
                                       preferred_element_type=jnp.float32)
        return jnp.concatenate([dg(ta), dg(tb)], axis=1)

    bp = bp_ref[...]
    bp2 = jnp.concatenate([bp, bp], axis=1)
    p2_ref[...] = proj(t2a_ref[...], t2b_ref[...], 0) + bp2
    p3_ref[...] = proj(t3a_ref[...], t3b_ref[...], 1)
    p4_ref[...] = proj(t4a_ref[...], t4b_ref[...], 2)


def _projected_tables(table_2, table_3, table_4, Wp, bp):
    rb = 2000
    grid = (V // 2) // rb
    top = pl.BlockSpec((rb, D), lambda i: (i, 0))
    bot = pl.BlockSpec((rb, D), lambda i: (i + grid, 0))
    return pl.pallas_call(
        _proj_body,
        grid=(grid,),
        in_specs=[top, bot, top, bot, top, bot,
                  pl.BlockSpec((D, 3 * D), lambda i: (0, 0)),
                  pl.BlockSpec((1, D), lambda i: (0, 0))],
        out_specs=[pl.BlockSpec((rb, 2 * D), lambda i: (i, 0))] * 3,
        out_shape=[jax.ShapeDtypeStruct((V // 2, 2 * D), jnp.float32)] * 3,
    )(table_2, table_2, table_3, table_3, table_4, table_4, Wp,
      bp.reshape(1, D))


def _sc_body(p2, p3, p4, h2, h3, h4, bp_hbm, out_hbm,
             idx2, idx3, idx4, outp, outp2, buf2, buf3, buf4,
             buf2b, buf3b, buf4b, biasv, semA, semB, semOA, semOB):
    wid = lax.axis_index("s") * NC + lax.axis_index("c")
    base = wid * BPW
    pltpu.sync_copy(h2.at[pl.ds(base, BPW)], idx2)
    pltpu.sync_copy(h3.at[pl.ds(base, BPW)], idx3)
    pltpu.sync_copy(h4.at[pl.ds(base, BPW)], idx4)
    pltpu.sync_copy(bp_hbm, biasv)
    for j in range(4):
        outp[0, pl.ds(L * j, L)] = biasv[pl.ds(L * j, L)]
        outp2[0, pl.ds(L * j, L)] = biasv[pl.ds(L * j, L)]

    def descs(r, b2, b3, b4, s):
        return (
            pltpu.make_async_copy(p2.at[idx2.at[r, 0]],
                                  b2.at[pl.ds(0, 128)], s),
            pltpu.make_async_copy(p2.at[idx2.at[r, 1, pl.ds(0, 72)]],
                                  b2.at[pl.ds(128, 72)], s),
            pltpu.make_async_copy(p3.at[idx3.at[r, 0]],
                                  b3.at[pl.ds(0, 128)], s),
            pltpu.make_async_copy(p3.at[idx3.at[r, 1, pl.ds(0, 72)]],
                                  b3.at[pl.ds(128, 72)], s),
            pltpu.make_async_copy(p4.at[idx4.at[r, 0]],
                                  b4.at[pl.ds(0, 128)], s),
            pltpu.make_async_copy(p4.at[idx4.at[r, 1, pl.ds(0, 72)]],
                                  b4.at[pl.ds(128, 72)], s),
        )

    def fire(r, b2, b3, b4, s):
        for c in descs(r, b2, b3, b4, s):
            c.start()

    def drain(r, b2, b3, b4, s):
        for c in descs(r, b2, b3, b4, s):
            c.wait()

    def alu(b2, b3, b4, op):
        for j in range(4):
            op[0, pl.ds(D + L * j, L)] = (b2[0, pl.ds(L * j, L)]
                                          + b3[0, pl.ds(L * j, L)])

        def pair_step(t, carry2):
            for j in range(4):
                op[t, pl.ds(L * j, L)] = (
                    b2[2 * t - 1, pl.ds(L * j, L)]
                    + b3[2 * t - 1, pl.ds(L * j, L)]
                    + b4[2 * t - 2, pl.ds(L * j, L)])
            for j in range(4):
                op[t, pl.ds(D + L * j, L)] = (
                    b2[2 * t, pl.ds(L * j, L)]
                    + b3[2 * t, pl.ds(L * j, L)]
                    + b4[2 * t - 1, pl.ds(L * j, L)])
            return carry2

        lax.fori_loop(1, 99, pair_step, 0, unroll=4)
        for j in range(4):
            op[99, pl.ds(L * j, L)] = (b2[197, pl.ds(L * j, L)]
                                       + b3[197, pl.ds(L * j, L)]
                                       + b4[196, pl.ds(L * j, L)])
        for j in range(4):
            op[99, pl.ds(D + L * j, L)] = b2[198, pl.ds(L * j, L)]

    def write_desc(r, op, s):
        return pltpu.make_async_copy(op.at[pl.ds(0, 100)],
                                     out_hbm.at[base + r], s)

    def write_out(r, op):
        pltpu.sync_copy(op.at[pl.ds(0, 100)], out_hbm.at[base + r])

    bufA = (buf2, buf3, buf4)
    bufB = (buf2b, buf3b, buf4b)
    fire(0, *bufA, semA)
    fire(1, *bufB, semB)

    def pipe_step(i, carry):
        ra = 2 * i
        drain(ra, *bufA, semA)

        @pl.when(i > 0)
        def _():
            write_desc(ra, outp, semOA).wait()

        alu(*bufA, outp)
        fire(lax.min(ra + 2, BPW - 1), *bufA, semA)
        write_desc(ra, outp, semOA).start()
        drain(ra + 1, *bufB, semB)

        @pl.when(i > 0)
        def _():
            write_desc(ra + 1, outp2, semOB).wait()

        alu(*bufB, outp2)
        fire(lax.min(ra + 3, BPW - 1), *bufB, semB)
        write_desc(ra + 1, outp2, semOB).start()
        return carry

    lax.fori_loop(0, BPW // 2 - 1, pipe_step, 0)
    drain(BPW - 2, *bufA, semA)
    write_desc(BPW - 2, outp, semOA).wait()
    alu(*bufA, outp)
    write_out(BPW - 2, outp)
    drain(BPW - 1, *bufB, semB)
    write_desc(BPW - 1, outp2, semOB).wait()
    alu(*bufB, outp2)
    write_out(BPW - 1, outp2)


def _sc_gather(P2, P3, P4, h2, h3, h4, bp):
    mesh = plsc.VectorSubcoreMesh(core_axis_name="c", subcore_axis_name="s",
                                  num_cores=NC, num_subcores=NS)
    f = pl.kernel(
        _sc_body,
        out_type=jax.ShapeDtypeStruct((B, 100, 2 * D), jnp.float32),
        mesh=mesh,
        compiler_params=pltpu.CompilerParams(use_tc_tiling_on_sc=False),
        scratch_types=[
            pltpu.VMEM((BPW, 2, 128), jnp.int32),
            pltpu.VMEM((BPW, 2, 128), jnp.int32),
            pltpu.VMEM((BPW, 2, 128), jnp.int32),
            pltpu.VMEM((104, 2 * D), jnp.float32),
            pltpu.VMEM((104, 2 * D), jnp.float32),
            pltpu.VMEM((200, D), jnp.float32),
            pltpu.VMEM((200, D), jnp.float32),
            pltpu.VMEM((200, D), jnp.float32),
            pltpu.VMEM((200, D), jnp.float32),
            pltpu.VMEM((200, D), jnp.float32),
            pltpu.VMEM((200, D), jnp.float32),
            pltpu.VMEM((D,), jnp.float32),
            pltpu.SemaphoreType.DMA,
            pltpu.SemaphoreType.DMA,
            pltpu.SemaphoreType.DMA,
            pltpu.SemaphoreType.DMA,
        ],
    )
    return f(P2, P3, P4, h2, h3, h4, bp)


def kernel(byte_sequence, table_2, table_3, table_4, Wp, bp):
    h2, h3, h4 = _hashes(byte_sequence)
    P2, P3, P4 = _projected_tables(table_2, table_3, table_4, Wp, bp)
    acc = _sc_gather(
        P2.reshape(V, D), P3.reshape(V, D), P4.reshape(V, D),
        h2.reshape(B, 2, 128), h3.reshape(B, 2, 128), h4.reshape(B, 2, 128),
        bp)
    return acc.reshape(B, S, D)
